# baseline (speedup 1.0000x reference)
import jax
import jax.numpy as jnp
from jax import lax
from jax.experimental import pallas as pl
from jax.experimental.pallas import tpu as pltpu
from jax.experimental.pallas import tpu_sc as plsc

P_DIM = 128
E = 320000
NC, NS, L = 2, 16, 16
NW = NC * NS
RPW = E // NW
CH = 400
NCH = RPW // CH
CHW = CH * P_DIM
RU = 8
AUX_R = E // P_DIM

_DN_COL = (((0,), (1,)), ((), ()))
_DN_ROW = (((1,), (1,)), ((), ()))


def _sc_body(mu_hbm, mu_out, buf, accv, sem0, sem1):
    wid = lax.axis_index("s") * NC + lax.axis_index("c")
    base = wid * (RPW * P_DIM)
    sems = (sem0, sem1)

    def start(c):
        return pltpu.async_copy(
            mu_hbm.at[pl.ds(base + c * CHW, CHW)],
            buf.at[pl.ds((c % 2) * CHW, CHW)],
            sems[c % 2])

    pend = start(0)
    accs = tuple(jnp.zeros((L,), jnp.float32) for _ in range(8))
    for c in range(NCH):
        nxt = start(c + 1) if c + 1 < NCH else None
        pend.wait()
        pend = nxt
        b0 = (c % 2) * CHW

        def row_body(i, acc, b0=b0):
            off = b0 + i * (RU * P_DIM)
            a = list(acc)
            for u in range(RU):
                for k in range(8):
                    a[k] = a[k] + buf[pl.ds(off + u * P_DIM + k * L, L)]
            return tuple(a)

        accs = lax.fori_loop(0, CH // RU, row_body, accs)
    for k in range(8):
        accv[pl.ds(k * L, L)] = accs[k]
    pltpu.sync_copy(accv, mu_out.at[wid])


_sc_call = pl.kernel(
    _sc_body,
    out_type=jax.ShapeDtypeStruct((NW, P_DIM), jnp.float32),
    mesh=plsc.VectorSubcoreMesh(core_axis_name="c", subcore_axis_name="s",
                                num_cores=NC, num_subcores=NS),
    scratch_types=[
        pltpu.VMEM((2 * CHW,), jnp.float32),
        pltpu.VMEM((P_DIM,), jnp.float32),
        pltpu.SemaphoreType.DMA,
        pltpu.SemaphoreType.DMA,
    ],
)


def _aux_body(a0_ref, a1_ref, a2_ref, a3_ref, w2_ref, w3_ref, w4_ref,
              w5_ref, w6_ref, w7_ref, w8_ref, w9_ref, w10_ref, xi_ref,
              out_ref):
    tmp = lax.dot_general(xi_ref[...], w10_ref[...], _DN_ROW,
                          preferred_element_type=jnp.float32,
                          precision=lax.Precision.HIGHEST)
    planes = (a0_ref, a1_ref, a2_ref, a3_ref)
    wvs = (w3_ref, w5_ref, w7_ref, w9_ref)
    wms = (w2_ref, w4_ref, w6_ref, w8_ref)
    for c in range(4):
        x = planes[c][...]
        p = jnp.sum(jnp.maximum(x, 0.0))
        n = jnp.sum(jnp.maximum(-x, 0.0))
        wv = wvs[c][...]
        rp = lax.dot_general(jnp.maximum(wv, 0.0), wms[c][...], _DN_COL,
                             preferred_element_type=jnp.float32,
                          precision=lax.Precision.HIGHEST)
        rn = lax.dot_general(jnp.maximum(-wv, 0.0), wms[c][...], _DN_COL,
                             preferred_element_type=jnp.float32,
                          precision=lax.Precision.HIGHEST)
        tmp += p * rp + n * rn
    out_ref[...] = tmp


def _fin_body(pm_ref, base_ref, w1_ref, out_ref):
    s = jnp.sum(pm_ref[...], axis=0, keepdims=True)
    tmp = lax.dot_general(s, w1_ref[...], _DN_ROW,
                          preferred_element_type=jnp.float32,
                          precision=lax.Precision.HIGHEST)
    out_ref[...] = jnp.maximum(tmp + base_ref[...], 0.0)


def _full(shape):
    return pl.BlockSpec(shape, lambda: (0,) * len(shape))


def kernel(xi, mu_N, h, hc, s, sc, W1, W2, W3, W4, W5, W6, W7, W8, W9, W10):
    part_mu = _sc_call(mu_N.reshape(-1))

    planes = [x.reshape(AUX_R, P_DIM) for x in (h, hc, s, sc)]
    base = pl.pallas_call(
        _aux_body,
        in_specs=[_full((AUX_R, P_DIM))] * 4
        + [_full((P_DIM, P_DIM)), _full((P_DIM, 1)), _full((P_DIM, P_DIM)),
           _full((P_DIM, 1)), _full((P_DIM, P_DIM)), _full((P_DIM, 1)),
           _full((P_DIM, P_DIM)), _full((P_DIM, 1)), _full((P_DIM, 2)),
           _full((1, 2))],
        out_specs=_full((1, P_DIM)),
        out_shape=jax.ShapeDtypeStruct((1, P_DIM), jnp.float32),
    )(*planes, W2, W3, W4, W5, W6, W7, W8, W9, W10, xi.reshape(1, 2))

    out = pl.pallas_call(
        _fin_body,
        in_specs=[_full((NW, P_DIM)), _full((1, P_DIM)),
                  _full((P_DIM, P_DIM))],
        out_specs=_full((1, P_DIM)),
        out_shape=jax.ShapeDtypeStruct((1, P_DIM), jnp.float32),
    )(part_mu, base, W1)
    return out.reshape(P_DIM)


# speedup vs baseline: 1.0114x; 1.0114x over previous
import jax
import jax.numpy as jnp
from jax import lax
from jax.experimental import pallas as pl
from jax.experimental.pallas import tpu as pltpu
from jax.experimental.pallas import tpu_sc as plsc

P_DIM = 128
E = 320000
NC, NS, L = 2, 16, 16
NW = NC * NS
RPW = 9600
E_SC = NW * RPW
E_TC = E - E_SC
CH = 400
NCH = RPW // CH
CHW = CH * P_DIM
RU = 8
AUX_R = E // P_DIM
TC_BR = 1600
TC_G = E_TC // TC_BR
MU_BLOCKS0 = E_SC // TC_BR


def _sc_body(mu_hbm, mu_out, buf, accv, sem0, sem1):
    wid = lax.axis_index("s") * NC + lax.axis_index("c")
    base = wid * (RPW * P_DIM)
    sems = (sem0, sem1)

    def start(c):
        return pltpu.async_copy(
            mu_hbm.at[pl.ds(base + c * CHW, CHW)],
            buf.at[pl.ds((c % 2) * CHW, CHW)],
            sems[c % 2])

    pend = start(0)
    accs = tuple(jnp.zeros((L,), jnp.float32) for _ in range(8))
    for c in range(NCH):
        nxt = start(c + 1) if c + 1 < NCH else None
        pend.wait()
        pend = nxt
        b0 = (c % 2) * CHW

        def row_body(i, acc, b0=b0):
            off = b0 + i * (RU * P_DIM)
            a = list(acc)
            for u in range(RU):
                for k in range(8):
                    a[k] = a[k] + buf[pl.ds(off + u * P_DIM + k * L, L)]
            return tuple(a)

        accs = lax.fori_loop(0, CH // RU, row_body, accs)
    for k in range(8):
        accv[pl.ds(k * L, L)] = accs[k]
    pltpu.sync_copy(accv, mu_out.at[wid])


def _sc_call(mu_flat):
    return pl.kernel(
        _sc_body,
        out_type=jax.ShapeDtypeStruct((NW, P_DIM), jnp.float32),
        mesh=plsc.VectorSubcoreMesh(core_axis_name="c", subcore_axis_name="s",
                                    num_cores=NC, num_subcores=NS),
        scratch_types=[
            pltpu.VMEM((2 * CHW,), jnp.float32),
            pltpu.VMEM((P_DIM,), jnp.float32),
            pltpu.SemaphoreType.DMA,
            pltpu.SemaphoreType.DMA,
        ],
    )(mu_flat)


def _aux_body(mu_ref, a0_ref, a1_ref, a2_ref, a3_ref, w2t_ref, w4t_ref,
              w6t_ref, w8t_ref, wv_ref, w10t_ref, xi_ref,
              base_ref, stc_ref, acc_ref):
    k = pl.program_id(0)

    @pl.when(k == 0)
    def _init():
        acc_ref[...] = jnp.zeros_like(acc_ref)

    blk = mu_ref[...]
    acc_ref[...] += jnp.sum(blk.reshape(TC_BR // 8, 8, P_DIM), axis=0)

    @pl.when(k == TC_G - 1)
    def _finish():
        stc_ref[...] = acc_ref[...]
        tmp = jnp.dot(xi_ref[...], w10t_ref[...],
                      preferred_element_type=jnp.float32)
        planes = (a0_ref, a1_ref, a2_ref, a3_ref)
        wts = (w2t_ref, w4t_ref, w6t_ref, w8t_ref)
        wv = wv_ref[...]
        for c in range(4):
            x = planes[c][...]
            p = jnp.sum(jnp.maximum(x, 0.0))
            n = jnp.sum(jnp.maximum(-x, 0.0))
            v = (p * jnp.maximum(wv[c:c + 1], 0.0)
                 + n * jnp.maximum(-wv[c:c + 1], 0.0))
            tmp += jnp.dot(v, wts[c][...],
                           preferred_element_type=jnp.float32)
        base_ref[...] = tmp


def _fin_body(pm_ref, stc_ref, base_ref, w1t_ref, out_ref):
    s = (jnp.sum(pm_ref[...], axis=0, keepdims=True)
         + jnp.sum(stc_ref[...], axis=0, keepdims=True))
    tmp = jnp.dot(s, w1t_ref[...], preferred_element_type=jnp.float32)
    out_ref[...] = jnp.maximum(tmp + base_ref[...], 0.0)


def _full(shape):
    return pl.BlockSpec(shape, lambda k: (0,) * len(shape))


def _full0(shape):
    return pl.BlockSpec(shape, lambda: (0,) * len(shape))


def kernel(xi, mu_N, h, hc, s, sc, W1, W2, W3, W4, W5, W6, W7, W8, W9, W10):
    part_mu = _sc_call(mu_N.reshape(-1))

    planes = [x.reshape(AUX_R, P_DIM) for x in (h, hc, s, sc)]
    wv = jnp.stack([W3[:, 0], W5[:, 0], W7[:, 0], W9[:, 0]])
    base, s_tc = pl.pallas_call(
        _aux_body,
        grid=(TC_G,),
        in_specs=[pl.BlockSpec((TC_BR, P_DIM),
                               lambda k: (MU_BLOCKS0 + k, 0))]
        + [_full((AUX_R, P_DIM))] * 4
        + [_full((P_DIM, P_DIM))] * 4
        + [_full((4, P_DIM)), _full((2, P_DIM)), _full((1, 2))],
        out_specs=[_full((1, P_DIM)), _full((8, P_DIM))],
        out_shape=[jax.ShapeDtypeStruct((1, P_DIM), jnp.float32),
                   jax.ShapeDtypeStruct((8, P_DIM), jnp.float32)],
        scratch_shapes=[pltpu.VMEM((8, P_DIM), jnp.float32)],
        compiler_params=pltpu.CompilerParams(
            dimension_semantics=("arbitrary",)),
    )(mu_N, *planes, W2.T, W4.T, W6.T, W8.T, wv, W10.T, xi.reshape(1, 2))

    out = pl.pallas_call(
        _fin_body,
        in_specs=[_full0((NW, P_DIM)), _full0((8, P_DIM)),
                  _full0((1, P_DIM)), _full0((P_DIM, P_DIM))],
        out_specs=_full0((1, P_DIM)),
        out_shape=jax.ShapeDtypeStruct((1, P_DIM), jnp.float32),
    )(part_mu, s_tc, base, W1.T)
    return out.reshape(P_DIM)


# speedup vs baseline: 1.0724x; 1.0604x over previous
import jax
import jax.numpy as jnp
from jax import lax
from jax.experimental import pallas as pl
from jax.experimental.pallas import tpu as pltpu
from jax.experimental.pallas import tpu_sc as plsc

P_DIM = 128
E = 320000
NC, NS, L = 2, 16, 16
NW = NC * NS
RPW = 9375
E_SC = NW * RPW
E_TC = E - E_SC
CH = 375
NCH = RPW // CH
CHW = CH * P_DIM
RU = 5
AUX_R = E // P_DIM
TC_BR = 5000
TC_G = E_TC // TC_BR
MU_BLK0 = E_SC // TC_BR


def _sc_body(mu_hbm, mu_out, buf, accv, sem0, sem1):
    wid = lax.axis_index("s") * NC + lax.axis_index("c")
    base = wid * (RPW * P_DIM)
    sems = (sem0, sem1)

    def start(c):
        return pltpu.async_copy(
            mu_hbm.at[pl.ds(base + c * CHW, CHW)],
            buf.at[pl.ds((c % 2) * CHW, CHW)],
            sems[c % 2])

    pend = start(0)
    accs = tuple(jnp.zeros((L,), jnp.float32) for _ in range(8))
    for c in range(NCH):
        nxt = start(c + 1) if c + 1 < NCH else None
        pend.wait()
        pend = nxt
        b0 = (c % 2) * CHW

        def row_body(i, acc, b0=b0):
            off = b0 + i * (RU * P_DIM)
            a = list(acc)
            for u in range(RU):
                for k in range(8):
                    a[k] = a[k] + buf[pl.ds(off + u * P_DIM + k * L, L)]
            return tuple(a)

        accs = lax.fori_loop(0, CH // RU, row_body, accs)
    for k in range(8):
        accv[pl.ds(k * L, L)] = accs[k]
    pltpu.sync_copy(accv, mu_out.at[wid])


def _sc_call(mu_flat):
    return pl.kernel(
        _sc_body,
        out_type=jax.ShapeDtypeStruct((NW, P_DIM), jnp.float32),
        mesh=plsc.VectorSubcoreMesh(core_axis_name="c", subcore_axis_name="s",
                                    num_cores=NC, num_subcores=NS),
        scratch_types=[
            pltpu.VMEM((2 * CHW,), jnp.float32),
            pltpu.VMEM((P_DIM,), jnp.float32),
            pltpu.SemaphoreType.DMA,
            pltpu.SemaphoreType.DMA,
        ],
    )(mu_flat)


def _mu_tc_body(mu_ref, stc_ref, acc_ref):
    k = pl.program_id(0)

    @pl.when(k == 0)
    def _init():
        acc_ref[...] = jnp.zeros_like(acc_ref)

    blk = mu_ref[...]
    acc_ref[...] += jnp.sum(blk.reshape(TC_BR // 8, 8, P_DIM), axis=0)

    @pl.when(k == TC_G - 1)
    def _done():
        stc_ref[...] = acc_ref[...]


def _aux_body(a0_ref, a1_ref, a2_ref, a3_ref, w2t_ref, w4t_ref, w6t_ref,
              w8t_ref, wv_ref, w10t_ref, xi_ref, base_ref):
    tmp = jnp.dot(xi_ref[...], w10t_ref[...],
                  preferred_element_type=jnp.float32)
    planes = (a0_ref, a1_ref, a2_ref, a3_ref)
    wts = (w2t_ref, w4t_ref, w6t_ref, w8t_ref)
    wv = wv_ref[...]
    for c in range(4):
        x = planes[c][...]
        p = jnp.sum(jnp.maximum(x, 0.0))
        n = jnp.sum(jnp.maximum(-x, 0.0))
        v = (p * jnp.maximum(wv[c:c + 1], 0.0)
             + n * jnp.maximum(-wv[c:c + 1], 0.0))
        tmp += jnp.dot(v, wts[c][...], preferred_element_type=jnp.float32)
    base_ref[...] = tmp


def _fin_body(pm_ref, stc_ref, base_ref, w1t_ref, out_ref):
    s = (jnp.sum(pm_ref[...], axis=0, keepdims=True)
         + jnp.sum(stc_ref[...], axis=0, keepdims=True))
    tmp = jnp.dot(s, w1t_ref[...], preferred_element_type=jnp.float32)
    out_ref[...] = jnp.maximum(tmp + base_ref[...], 0.0)


def _full(shape):
    return pl.BlockSpec(shape, lambda k: (0,) * len(shape))


def _full0(shape):
    return pl.BlockSpec(shape, lambda: (0,) * len(shape))


def kernel(xi, mu_N, h, hc, s, sc, W1, W2, W3, W4, W5, W6, W7, W8, W9, W10):
    part_mu = _sc_call(mu_N.reshape(-1))

    s_tc = pl.pallas_call(
        _mu_tc_body,
        grid=(TC_G,),
        in_specs=[pl.BlockSpec((TC_BR, P_DIM), lambda k: (MU_BLK0 + k, 0))],
        out_specs=_full((8, P_DIM)),
        out_shape=jax.ShapeDtypeStruct((8, P_DIM), jnp.float32),
        scratch_shapes=[pltpu.VMEM((8, P_DIM), jnp.float32)],
        compiler_params=pltpu.CompilerParams(
            dimension_semantics=("arbitrary",)),
    )(mu_N)

    planes = [x.reshape(AUX_R, P_DIM) for x in (h, hc, s, sc)]
    wv = jnp.stack([W3[:, 0], W5[:, 0], W7[:, 0], W9[:, 0]])
    base = pl.pallas_call(
        _aux_body,
        in_specs=[_full0((AUX_R, P_DIM))] * 4
        + [_full0((P_DIM, P_DIM))] * 4
        + [_full0((4, P_DIM)), _full0((2, P_DIM)), _full0((1, 2))],
        out_specs=_full0((1, P_DIM)),
        out_shape=jax.ShapeDtypeStruct((1, P_DIM), jnp.float32),
    )(*planes, W2.T, W4.T, W6.T, W8.T, wv, W10.T, xi.reshape(1, 2))

    out = pl.pallas_call(
        _fin_body,
        in_specs=[_full0((NW, P_DIM)), _full0((8, P_DIM)),
                  _full0((1, P_DIM)), _full0((P_DIM, P_DIM))],
        out_specs=_full0((1, P_DIM)),
        out_shape=jax.ShapeDtypeStruct((1, P_DIM), jnp.float32),
    )(part_mu, s_tc, base, W1.T)
    return out.reshape(P_DIM)


# speedup vs baseline: 1.0742x; 1.0017x over previous
import jax
import jax.numpy as jnp
from jax import lax
from jax.experimental import pallas as pl
from jax.experimental.pallas import tpu as pltpu
from jax.experimental.pallas import tpu_sc as plsc

P_DIM = 128
E = 320000
NC, NS, L = 2, 16, 16
NW = NC * NS
RPW = 9375
E_SC = NW * RPW
E_TC = E - E_SC
CH = 375
NCH = RPW // CH
CHW = CH * P_DIM
RU = 5
AUX_R = E // P_DIM
TC_BR = 10000
TC_G = E_TC // TC_BR
MU_BLK0 = E_SC // TC_BR


def _sc_body(mu_hbm, mu_out, buf, accv, sem0, sem1):
    wid = lax.axis_index("s") * NC + lax.axis_index("c")
    base = wid * (RPW * P_DIM)
    sems = (sem0, sem1)

    def start(c):
        return pltpu.async_copy(
            mu_hbm.at[pl.ds(base + c * CHW, CHW)],
            buf.at[pl.ds((c % 2) * CHW, CHW)],
            sems[c % 2])

    pend = start(0)
    accs = tuple(jnp.zeros((L,), jnp.float32) for _ in range(8))
    for c in range(NCH):
        nxt = start(c + 1) if c + 1 < NCH else None
        pend.wait()
        pend = nxt
        b0 = (c % 2) * CHW

        def row_body(i, acc, b0=b0):
            off = b0 + i * (RU * P_DIM)
            a = list(acc)
            for u in range(RU):
                for k in range(8):
                    a[k] = a[k] + buf[pl.ds(off + u * P_DIM + k * L, L)]
            return tuple(a)

        accs = lax.fori_loop(0, CH // RU, row_body, accs)
    for k in range(8):
        accv[pl.ds(k * L, L)] = accs[k]
    pltpu.sync_copy(accv, mu_out.at[wid])


def _sc_call(mu_flat):
    return pl.kernel(
        _sc_body,
        out_type=jax.ShapeDtypeStruct((NW, P_DIM), jnp.float32),
        mesh=plsc.VectorSubcoreMesh(core_axis_name="c", subcore_axis_name="s",
                                    num_cores=NC, num_subcores=NS),
        scratch_types=[
            pltpu.VMEM((2 * CHW,), jnp.float32),
            pltpu.VMEM((P_DIM,), jnp.float32),
            pltpu.SemaphoreType.DMA,
            pltpu.SemaphoreType.DMA,
        ],
    )(mu_flat)


def _mu_tc_body(mu_ref, stc_ref, acc_ref):
    k = pl.program_id(0)

    @pl.when(k == 0)
    def _init():
        acc_ref[...] = jnp.zeros_like(acc_ref)

    blk = mu_ref[...]
    acc_ref[...] += jnp.sum(blk.reshape(TC_BR // 8, 8, P_DIM), axis=0)

    @pl.when(k == TC_G - 1)
    def _done():
        stc_ref[...] = acc_ref[...]


def _aux_body(a0_ref, a1_ref, a2_ref, a3_ref, w2t_ref, w4t_ref, w6t_ref,
              w8t_ref, wv_ref, w10t_ref, xi_ref, base_ref):
    tmp = jnp.dot(xi_ref[...], w10t_ref[...],
                  preferred_element_type=jnp.float32)
    planes = (a0_ref, a1_ref, a2_ref, a3_ref)
    wts = (w2t_ref, w4t_ref, w6t_ref, w8t_ref)
    wv = wv_ref[...]
    for c in range(4):
        x = planes[c][...]
        p = jnp.sum(jnp.maximum(x, 0.0))
        n = jnp.sum(jnp.maximum(-x, 0.0))
        v = (p * jnp.maximum(wv[c:c + 1], 0.0)
             + n * jnp.maximum(-wv[c:c + 1], 0.0))
        tmp += jnp.dot(v, wts[c][...], preferred_element_type=jnp.float32)
    base_ref[...] = tmp


def _fin_body(pm_ref, stc_ref, base_ref, w1t_ref, out_ref):
    s = (jnp.sum(pm_ref[...], axis=0, keepdims=True)
         + jnp.sum(stc_ref[...], axis=0, keepdims=True))
    tmp = jnp.dot(s, w1t_ref[...], preferred_element_type=jnp.float32)
    out_ref[...] = jnp.maximum(tmp + base_ref[...], 0.0)


def _full(shape):
    return pl.BlockSpec(shape, lambda k: (0,) * len(shape))


def _full0(shape):
    return pl.BlockSpec(shape, lambda: (0,) * len(shape))


def kernel(xi, mu_N, h, hc, s, sc, W1, W2, W3, W4, W5, W6, W7, W8, W9, W10):
    part_mu = _sc_call(mu_N.reshape(-1))

    s_tc = pl.pallas_call(
        _mu_tc_body,
        grid=(TC_G,),
        in_specs=[pl.BlockSpec((TC_BR, P_DIM), lambda k: (MU_BLK0 + k, 0))],
        out_specs=_full((8, P_DIM)),
        out_shape=jax.ShapeDtypeStruct((8, P_DIM), jnp.float32),
        scratch_shapes=[pltpu.VMEM((8, P_DIM), jnp.float32)],
        compiler_params=pltpu.CompilerParams(
            dimension_semantics=("arbitrary",)),
    )(mu_N)

    planes = [x.reshape(AUX_R, P_DIM) for x in (h, hc, s, sc)]
    wv = jnp.stack([W3[:, 0], W5[:, 0], W7[:, 0], W9[:, 0]])
    base = pl.pallas_call(
        _aux_body,
        in_specs=[_full0((AUX_R, P_DIM))] * 4
        + [_full0((P_DIM, P_DIM))] * 4
        + [_full0((4, P_DIM)), _full0((2, P_DIM)), _full0((1, 2))],
        out_specs=_full0((1, P_DIM)),
        out_shape=jax.ShapeDtypeStruct((1, P_DIM), jnp.float32),
    )(*planes, W2.T, W4.T, W6.T, W8.T, wv, W10.T, xi.reshape(1, 2))

    out = pl.pallas_call(
        _fin_body,
        in_specs=[_full0((NW, P_DIM)), _full0((8, P_DIM)),
                  _full0((1, P_DIM)), _full0((P_DIM, P_DIM))],
        out_specs=_full0((1, P_DIM)),
        out_shape=jax.ShapeDtypeStruct((1, P_DIM), jnp.float32),
    )(part_mu, s_tc, base, W1.T)
    return out.reshape(P_DIM)
